# baseline (speedup 1.0000x reference)
import functools

import jax
import jax.numpy as jnp
from jax import lax
from jax.experimental import pallas as pl
from jax.experimental.pallas import tpu as pltpu
from jax.experimental.pallas import tpu_sc as plsc


def _argmin_latents(x, x2r, e2c, emb, KT):
    B, D, L = x.shape
    K = emb.shape[0]
    NT = K // KT

    def body(x_ref, e2_ref, x2_ref, emb_ref, out_ref):
        xb = x_ref[0] * (-2.0)
        x2b = x2_ref[0]
        NB = 8
        RB = KT // NB
        li = lax.broadcasted_iota(jnp.int32, (RB, L), 0).astype(jnp.float32)

        run_v = jnp.full((1, L), jnp.inf, jnp.float32)
        run_i = jnp.zeros((1, L), jnp.float32)
        for t in range(NT):
            et = emb_ref[pl.ds(t * KT, KT), :]
            c2 = lax.dot_general(et, xb, (((1,), (0,)), ((), ())))
            e2t = e2_ref[pl.ds(t * KT, KT), :]
            dist = x2b + c2 + e2t
            blks = [dist[b * RB:(b + 1) * RB, :] for b in range(NB)]
            bms = [jnp.min(blk, axis=0, keepdims=True) for blk in blks]
            tv = bms[0]
            for b in range(1, NB):
                tv = jnp.minimum(tv, bms[b])
            wb = jnp.full((1, L), float(NB), jnp.float32)
            for b in range(NB - 1, -1, -1):
                wb = jnp.where(bms[b] == tv, float(b), wb)
            dwin = blks[NB - 1]
            for b in range(NB - 2, -1, -1):
                dwin = jnp.where(wb == float(b), blks[b], dwin)
            ti = (jnp.min(jnp.where(dwin == tv, li, float(K)), axis=0,
                          keepdims=True)
                  + wb * float(RB) + float(t * KT))
            upd = tv < run_v
            run_v = jnp.where(upd, tv, run_v)
            run_i = jnp.where(upd, ti, run_i)
        out_ref[0] = run_i.astype(jnp.int32)

    return pl.pallas_call(
        body,
        grid=(B,),
        in_specs=[
            pl.BlockSpec((1, D, L), lambda b: (b, 0, 0)),
            pl.BlockSpec((K, 1), lambda b: (0, 0)),
            pl.BlockSpec((1, 1, L), lambda b: (b, 0, 0)),
            pl.BlockSpec((K, D), lambda b: (0, 0)),
        ],
        out_specs=pl.BlockSpec((1, 1, L), lambda b: (b, 0, 0)),
        out_shape=jax.ShapeDtypeStruct((B, 1, L), jnp.int32),
    )(x, e2c, x2r, emb)


def _sc_gather(emb, idx):
    info = plsc.get_sparse_core_info()
    NC, NS = info.num_cores, info.num_subcores
    NW = NC * NS
    N = idx.shape[0]
    D = emb.shape[1]
    bpw = N // NW
    mesh = plsc.VectorSubcoreMesh(core_axis_name="c", subcore_axis_name="s")

    @functools.partial(
        pl.kernel,
        mesh=mesh,
        out_type=jax.ShapeDtypeStruct((N, D), jnp.float32),
        scratch_types=[
            pltpu.VMEM((bpw,), jnp.int32),
            pltpu.VMEM((bpw, D), jnp.float32),
            pltpu.SemaphoreType.DMA,
        ],
    )
    def gk(table_hbm, idx_hbm, out_hbm, idx_v, rows_v, sem):
        wid = lax.axis_index("s") * NC + lax.axis_index("c")
        base = wid * bpw
        pltpu.sync_copy(idx_hbm.at[pl.ds(base, bpw)], idx_v)
        pltpu.async_copy(table_hbm.at[idx_v], rows_v, sem).wait()
        pltpu.sync_copy(rows_v, out_hbm.at[pl.ds(base, bpw)])

    return gk(emb, idx)


def _transpose_out(q):
    B, L, D = q.shape

    def body(q_ref, o_ref, o2_ref):
        qt = q_ref[0].T
        o_ref[0] = qt
        o2_ref[0] = qt

    return pl.pallas_call(
        body,
        grid=(B,),
        in_specs=[pl.BlockSpec((1, L, D), lambda b: (b, 0, 0))],
        out_specs=[pl.BlockSpec((1, D, L), lambda b: (b, 0, 0)),
                   pl.BlockSpec((1, D, L), lambda b: (b, 0, 0))],
        out_shape=[jax.ShapeDtypeStruct((B, D, L), jnp.float32),
                   jax.ShapeDtypeStruct((B, D, L), jnp.float32)],
    )(q)


def kernel(input, embedding):
    B, D, L = input.shape
    K = embedding.shape[0]
    xT = jnp.transpose(input, (0, 2, 1))
    x2 = jnp.sum(xT * xT, axis=-1, keepdims=True)
    x2r = jnp.transpose(x2, (0, 2, 1))
    e2 = jnp.sum(embedding * embedding, axis=-1)
    lat = _argmin_latents(input, x2r, e2.reshape(K, 1), embedding, KT=512)
    idx = lat.reshape(B * L)
    q = _sc_gather(embedding, idx)
    out, out2 = _transpose_out(q.reshape(B, L, D))
    return (out, out2)


# speedup vs baseline: 1.0836x; 1.0836x over previous
import functools

import jax
import jax.numpy as jnp
from jax import lax
from jax.experimental import pallas as pl
from jax.experimental.pallas import tpu as pltpu
from jax.experimental.pallas import tpu_sc as plsc


def _argmin_latents(x, x2r, e2c, emb, KT):
    B, D, L = x.shape
    K = emb.shape[0]
    NT = K // KT

    def body(x_ref, e2_ref, x2_ref, emb_ref, out_ref):
        xb = x_ref[0] * (-2.0)
        x2b = x2_ref[0]
        NB = 8
        RB = KT // NB
        li = lax.broadcasted_iota(jnp.int32, (RB, L), 0).astype(jnp.float32)

        run_v = jnp.full((1, L), jnp.inf, jnp.float32)
        run_i = jnp.zeros((1, L), jnp.float32)
        for t in range(NT):
            et = emb_ref[pl.ds(t * KT, KT), :]
            c2 = lax.dot_general(et, xb, (((1,), (0,)), ((), ())))
            e2t = e2_ref[pl.ds(t * KT, KT), :]
            dist = x2b + c2 + e2t
            blks = [dist[b * RB:(b + 1) * RB, :] for b in range(NB)]
            bms = [jnp.min(blk, axis=0, keepdims=True) for blk in blks]
            tv = bms[0]
            for b in range(1, NB):
                tv = jnp.minimum(tv, bms[b])
            wb = jnp.full((1, L), float(NB), jnp.float32)
            for b in range(NB - 1, -1, -1):
                wb = jnp.where(bms[b] == tv, float(b), wb)
            dwin = blks[NB - 1]
            for b in range(NB - 2, -1, -1):
                dwin = jnp.where(wb == float(b), blks[b], dwin)
            ti = (jnp.min(jnp.where(dwin == tv, li, float(K)), axis=0,
                          keepdims=True)
                  + wb * float(RB) + float(t * KT))
            upd = tv < run_v
            run_v = jnp.where(upd, tv, run_v)
            run_i = jnp.where(upd, ti, run_i)
        out_ref[0] = run_i.astype(jnp.int32)

    return pl.pallas_call(
        body,
        grid=(B,),
        in_specs=[
            pl.BlockSpec((1, D, L), lambda b: (b, 0, 0)),
            pl.BlockSpec((K, 1), lambda b: (0, 0)),
            pl.BlockSpec((1, 1, L), lambda b: (b, 0, 0)),
            pl.BlockSpec((K, D), lambda b: (0, 0)),
        ],
        out_specs=pl.BlockSpec((1, 1, L), lambda b: (b, 0, 0)),
        out_shape=jax.ShapeDtypeStruct((B, 1, L), jnp.int32),
    )(x, e2c, x2r, emb)


def _sc_gather(emb, idx):
    info = plsc.get_sparse_core_info()
    NC, NS = info.num_cores, info.num_subcores
    NW = NC * NS
    N = idx.shape[0]
    D = emb.shape[1]
    bpw = N // NW
    mesh = plsc.VectorSubcoreMesh(core_axis_name="c", subcore_axis_name="s")

    @functools.partial(
        pl.kernel,
        mesh=mesh,
        out_type=jax.ShapeDtypeStruct((N, D), jnp.float32),
        scratch_types=[
            pltpu.VMEM((bpw,), jnp.int32),
            pltpu.VMEM((bpw, D), jnp.float32),
            pltpu.SemaphoreType.DMA,
        ],
    )
    def gk(table_hbm, idx_hbm, out_hbm, idx_v, rows_v, sem):
        wid = lax.axis_index("s") * NC + lax.axis_index("c")
        base = wid * bpw
        pltpu.sync_copy(idx_hbm.at[pl.ds(base, bpw)], idx_v)
        pltpu.async_copy(table_hbm.at[idx_v], rows_v, sem).wait()
        pltpu.sync_copy(rows_v, out_hbm.at[pl.ds(base, bpw)])

    return gk(emb, idx)


def _transpose_out(q):
    B, L, D = q.shape

    def body(q_ref, o_ref, o2_ref):
        qt = q_ref[0].T
        o_ref[0] = qt
        o2_ref[0] = qt

    return pl.pallas_call(
        body,
        grid=(B,),
        in_specs=[pl.BlockSpec((1, L, D), lambda b: (b, 0, 0))],
        out_specs=[pl.BlockSpec((1, D, L), lambda b: (b, 0, 0)),
                   pl.BlockSpec((1, D, L), lambda b: (b, 0, 0))],
        out_shape=[jax.ShapeDtypeStruct((B, D, L), jnp.float32),
                   jax.ShapeDtypeStruct((B, D, L), jnp.float32)],
    )(q)


def kernel(input, embedding):
    B, D, L = input.shape
    K = embedding.shape[0]
    xT = jnp.transpose(input, (0, 2, 1))
    x2 = jnp.sum(xT * xT, axis=-1, keepdims=True)
    x2r = jnp.transpose(x2, (0, 2, 1))
    e2 = jnp.sum(embedding * embedding, axis=-1)
    lat = _argmin_latents(input, x2r, e2.reshape(K, 1), embedding, KT=1024)
    idx = lat.reshape(B * L)
    q = _sc_gather(embedding, idx)
    out, out2 = _transpose_out(q.reshape(B, L, D))
    return (out, out2)
